# baseline (speedup 1.0000x reference)
import functools
import math

import jax
import jax.numpy as jnp
from jax import lax
from jax.experimental import pallas as pl
from jax.experimental.pallas import tpu as pltpu

_L = 128
_H = 128
_DPE = 128


def _sigmoid(x):
    return 0.5 * (jnp.tanh(0.5 * x) + 1.0)


def _pos_enc_pair():
    pos = jnp.arange(_L, dtype=jnp.float32)[:, None]
    div = jnp.exp(jnp.arange(0, _DPE, 2, dtype=jnp.float32) *
                  (-math.log(10000.0) / _DPE))
    pe = jnp.zeros((_L, _DPE), jnp.float32)
    pe = pe.at[:, 0::2].set(jnp.sin(pos * div))
    pe = pe.at[:, 1::2].set(jnp.cos(pos * div))
    return jnp.concatenate([pe, pe[::-1]], axis=1)


def _mix_dirs(wf, wb):
    n_in = wf.shape[1]
    wtf, wtb = wf.T, wb.T
    out = jnp.zeros((2 * n_in, 6 * _H), jnp.float32)
    for g in range(3):
        out = out.at[:n_in, (2 * g) * _H:(2 * g + 1) * _H].set(
            wtf[:, g * _H:(g + 1) * _H])
        out = out.at[n_in:, (2 * g + 1) * _H:(2 * g + 2) * _H].set(
            wtb[:, g * _H:(g + 1) * _H])
    return out


def _mix_bias(bf, bb):
    out = jnp.zeros((1, 6 * _H), jnp.float32)
    for g in range(3):
        out = out.at[0, (2 * g) * _H:(2 * g + 1) * _H].set(bf[g * _H:(g + 1) * _H])
        out = out.at[0, (2 * g + 1) * _H:(2 * g + 2) * _H].set(bb[g * _H:(g + 1) * _H])
    return out


def _weights_kernel(pe_ref, wih_ref, bih_ref, whh_ref, bhh_ref, lin_ref,
                    w_ref, h_scr):
    H2 = 2 * _H
    gi = jnp.dot(pe_ref[...], wih_ref[...],
                 preferred_element_type=jnp.float32) + bih_ref[...]
    whh = whh_ref[...]
    bhh = bhh_ref[...]
    h = jnp.zeros((1, H2), jnp.float32)
    for s in range(_L):
        gh = jnp.dot(h, whh, preferred_element_type=jnp.float32) + bhh
        g = gi[s:s + 1, :]
        r = _sigmoid(g[:, :H2] + gh[:, :H2])
        z = _sigmoid(g[:, H2:2 * H2] + gh[:, H2:2 * H2])
        n = jnp.tanh(g[:, 2 * H2:] + r * gh[:, 2 * H2:])
        h = n + z * (h - n)
        h_scr[pl.ds(s, 1), :_H] = h[:, :_H]
        h_scr[pl.ds(_L - 1 - s, 1), _H:] = h[:, _H:]
    logits = jnp.dot(h_scr[...], lin_ref[...],
                     preferred_element_type=jnp.float32)
    m = jnp.max(logits, axis=0, keepdims=True)
    e = jnp.exp(logits - m)
    w_ref[...] = e / jnp.sum(e, axis=0, keepdims=True)


def _sort_passes():
    lane = lax.broadcasted_iota(jnp.int32, (1, 128), 1)
    passes = []
    k = 2
    while k <= 128:
        j = k // 2
        while j >= 1:
            lower = (lane & j) == 0
            take_max = lower == ((lane & k) == 0)
            passes.append((j, lower, take_max))
            j //= 2
        k *= 2
    return passes


def _pool_kernel(w_ref, feat_ref, out_ref, sort_scr, *, tr, ch):
    passes = _sort_passes()

    def chunk_body(i, carry):
        x = feat_ref[pl.ds(i * ch, ch), :]
        for j, lower, take_max in passes:
            below = pltpu.roll(x, 128 - j, axis=1)
            above = pltpu.roll(x, j, axis=1)
            partner = jnp.where(lower, below, above)
            x = jnp.where(take_max, jnp.maximum(x, partner),
                          jnp.minimum(x, partner))
        sort_scr[pl.ds(i * ch, ch), :] = x
        return carry

    lax.fori_loop(0, tr // ch, chunk_body, 0)
    out_ref[...] = jnp.dot(sort_scr[...], w_ref[...],
                           preferred_element_type=jnp.float32)


def kernel(features, w_ih_f, w_hh_f, b_ih_f, b_hh_f,
           w_ih_b, w_hh_b, b_ih_b, b_hh_b, lin_w):
    B, K, D = features.shape
    assert D == _L
    n = B * K

    pe2 = _pos_enc_pair()
    wih = _mix_dirs(w_ih_f, w_ih_b)
    whh = _mix_dirs(w_hh_f, w_hh_b)
    bih = _mix_bias(b_ih_f, b_ih_b)
    bhh = _mix_bias(b_hh_f, b_hh_b)
    lin = (lin_w.reshape(_H) * (0.5 / 0.1)).astype(jnp.float32)
    lin2 = jnp.concatenate([lin, lin])[:, None]

    w_col = pl.pallas_call(
        _weights_kernel,
        out_shape=jax.ShapeDtypeStruct((_L, 1), jnp.float32),
        scratch_shapes=[pltpu.VMEM((_L, 2 * _H), jnp.float32)],
    )(pe2, wih, bih, whh, bhh, lin2)

    flat = features.reshape(n, D)
    tr = 4096 if n % 4096 == 0 else n
    ch = min(64, tr)
    grid = (n // tr,)
    out = pl.pallas_call(
        functools.partial(_pool_kernel, tr=tr, ch=ch),
        out_shape=jax.ShapeDtypeStruct((n, 1), jnp.float32),
        grid=grid,
        in_specs=[
            pl.BlockSpec((_L, 1), lambda i: (0, 0)),
            pl.BlockSpec((tr, D), lambda i: (i, 0)),
        ],
        out_specs=pl.BlockSpec((tr, 1), lambda i: (i, 0)),
        scratch_shapes=[pltpu.VMEM((tr, D), jnp.float32)],
        compiler_params=pltpu.CompilerParams(
            dimension_semantics=("parallel",),
            vmem_limit_bytes=32 * 1024 * 1024,
        ),
    )(w_col, flat)

    return out.reshape(B, K)


# speedup vs baseline: 9.7295x; 9.7295x over previous
import functools
import math

import numpy as np
import jax
import jax.numpy as jnp
from jax import lax
from jax.experimental import pallas as pl
from jax.experimental.pallas import tpu as pltpu

_L = 128
_H = 128
_DPE = 128


def _sigmoid(x):
    return 0.5 * (jnp.tanh(0.5 * x) + 1.0)


def _pos_enc_pair():
    pos = jnp.arange(_L, dtype=jnp.float32)[:, None]
    div = jnp.exp(jnp.arange(0, _DPE, 2, dtype=jnp.float32) *
                  (-math.log(10000.0) / _DPE))
    pe = jnp.zeros((_L, _DPE), jnp.float32)
    pe = pe.at[:, 0::2].set(jnp.sin(pos * div))
    pe = pe.at[:, 1::2].set(jnp.cos(pos * div))
    return jnp.concatenate([pe, pe[::-1]], axis=1)


def _mix_dirs(wf, wb):
    n_in = wf.shape[1]
    wtf, wtb = wf.T, wb.T
    out = jnp.zeros((2 * n_in, 6 * _H), jnp.float32)
    for g in range(3):
        out = out.at[:n_in, (2 * g) * _H:(2 * g + 1) * _H].set(
            wtf[:, g * _H:(g + 1) * _H])
        out = out.at[n_in:, (2 * g + 1) * _H:(2 * g + 2) * _H].set(
            wtb[:, g * _H:(g + 1) * _H])
    return out


def _mix_bias(bf, bb):
    out = jnp.zeros((1, 6 * _H), jnp.float32)
    for g in range(3):
        out = out.at[0, (2 * g) * _H:(2 * g + 1) * _H].set(bf[g * _H:(g + 1) * _H])
        out = out.at[0, (2 * g + 1) * _H:(2 * g + 2) * _H].set(bb[g * _H:(g + 1) * _H])
    return out


def _weights_kernel(pe_ref, wih_ref, bih_ref, whh_ref, bhh_ref, lin_ref,
                    w_ref, h_scr):
    H2 = 2 * _H
    gi = jnp.dot(pe_ref[...], wih_ref[...],
                 preferred_element_type=jnp.float32) + bih_ref[...]
    whh = whh_ref[...]
    bhh = bhh_ref[...]
    h = jnp.zeros((1, H2), jnp.float32)
    for s in range(_L):
        gh = jnp.dot(h, whh, preferred_element_type=jnp.float32) + bhh
        g = gi[s:s + 1, :]
        r = _sigmoid(g[:, :H2] + gh[:, :H2])
        z = _sigmoid(g[:, H2:2 * H2] + gh[:, H2:2 * H2])
        n = jnp.tanh(g[:, 2 * H2:] + r * gh[:, 2 * H2:])
        h = n + z * (h - n)
        h_scr[pl.ds(s, 1), :_H] = h[:, :_H]
        h_scr[pl.ds(_L - 1 - s, 1), _H:] = h[:, _H:]
    logits = jnp.dot(h_scr[...], lin_ref[...],
                     preferred_element_type=jnp.float32)
    m = jnp.max(logits, axis=0, keepdims=True)
    e = jnp.exp(logits - m)
    w_ref[...] = e / jnp.sum(e, axis=0, keepdims=True)


_SHIFTS = [k // 2 >> p for k in [2, 4, 8, 16, 32, 64, 128]
           for p in range(100) if k // 2 >> p >= 1]


def _sort_masks():
    lane = np.arange(128)
    rows = []
    k = 2
    while k <= 128:
        j = k // 2
        while j >= 1:
            lower = (lane & j) == 0
            take_max = lower == ((lane & k) == 0)
            rows.append(np.where(lower, 1.0, -1.0))
            rows.append(np.where(take_max, 1.0, -1.0))
            j //= 2
        k *= 2
    rows.extend([np.zeros(128)] * (64 - len(rows)))
    return jnp.asarray(np.stack(rows).astype(np.float32)).astype(jnp.bfloat16)


def _pool_kernel(w_ref, masks_ref, feat_ref, out_ref):
    x = feat_ref[...].astype(jnp.bfloat16)
    zero = jnp.zeros((), jnp.bfloat16)
    lane = lax.broadcasted_iota(jnp.int32, (1, 128), 1)
    for p, j in enumerate(_SHIFTS):
        take_max_f = masks_ref[2 * p + 1:2 * p + 2, :]
        xi = pltpu.bitcast(x, jnp.int32)
        partner_i = jnp.take_along_axis(
            xi, jnp.broadcast_to(lane ^ j, xi.shape), axis=1)
        partner = pltpu.bitcast(partner_i, jnp.bfloat16)
        x = jnp.where(take_max_f > zero, jnp.maximum(x, partner),
                      jnp.minimum(x, partner))
    out_ref[...] = jnp.dot(x, w_ref[...].astype(jnp.bfloat16),
                           preferred_element_type=jnp.float32)


def kernel(features, w_ih_f, w_hh_f, b_ih_f, b_hh_f,
           w_ih_b, w_hh_b, b_ih_b, b_hh_b, lin_w):
    B, K, D = features.shape
    assert D == _L
    n = B * K

    pe2 = _pos_enc_pair()
    wih = _mix_dirs(w_ih_f, w_ih_b)
    whh = _mix_dirs(w_hh_f, w_hh_b)
    bih = _mix_bias(b_ih_f, b_ih_b)
    bhh = _mix_bias(b_hh_f, b_hh_b)
    lin = (lin_w.reshape(_H) * (0.5 / 0.1)).astype(jnp.float32)
    lin2 = jnp.concatenate([lin, lin])[:, None]

    w_col = pl.pallas_call(
        _weights_kernel,
        out_shape=jax.ShapeDtypeStruct((_L, 1), jnp.float32),
        scratch_shapes=[pltpu.VMEM((_L, 2 * _H), jnp.float32)],
    )(pe2, wih, bih, whh, bhh, lin2)

    flat = features.reshape(n, D)
    tr = 2048 if n % 2048 == 0 else n
    grid = (n // tr,)
    out = pl.pallas_call(
        _pool_kernel,
        out_shape=jax.ShapeDtypeStruct((n, 1), jnp.float32),
        grid=grid,
        in_specs=[
            pl.BlockSpec((_L, 1), lambda i: (0, 0)),
            pl.BlockSpec((64, D), lambda i: (0, 0)),
            pl.BlockSpec((tr, D), lambda i: (i, 0)),
        ],
        out_specs=pl.BlockSpec((tr, 1), lambda i: (i, 0)),
        compiler_params=pltpu.CompilerParams(
            dimension_semantics=("parallel",),
            vmem_limit_bytes=32 * 1024 * 1024,
        ),
    )(w_col, _sort_masks(), flat)

    return out.reshape(B, K)
